# Initial kernel scaffold; baseline (speedup 1.0000x reference)
import jax
import jax.numpy as jnp
from jax.experimental import pallas as pl


def kernel(x, table):
    raise NotImplementedError("write your pallas kernel here")


# speedup vs baseline: 3.9546x; 3.9546x over previous
import functools

import jax
import jax.numpy as jnp
from jax import lax
from jax.experimental import pallas as pl
from jax.experimental.pallas import tpu as pltpu
from jax.experimental.pallas import tpu_sc as plsc


def _make_gather(T: int, N: int, D: int):
    info = plsc.get_sparse_core_info()
    NC, NS = info.num_cores, info.num_subcores
    NW = NC * NS
    assert T % NW == 0
    per_w = T // NW
    CHUNK = 512
    assert per_w % CHUNK == 0
    n_chunks = per_w // CHUNK

    mesh = plsc.VectorSubcoreMesh(core_axis_name="c", subcore_axis_name="s")

    @functools.partial(
        pl.kernel,
        mesh=mesh,
        out_type=jax.ShapeDtypeStruct((T, D), jnp.float32),
        scratch_types=[
            pltpu.VMEM((CHUNK,), jnp.int32),
            pltpu.VMEM((CHUNK, D), jnp.float32),
            pltpu.SemaphoreType.DMA,
        ],
        compiler_params=pltpu.CompilerParams(use_tc_tiling_on_sc=False),
    )
    def gather_kernel(table_hbm, idx_hbm, out_hbm, idx_v, rows_v, sem):
        wid = lax.axis_index("s") * NC + lax.axis_index("c")
        base = wid * per_w

        def body(i, _):
            off = base + i * CHUNK
            pltpu.sync_copy(idx_hbm.at[pl.ds(off, CHUNK)], idx_v)
            pltpu.async_copy(table_hbm.at[idx_v], rows_v, sem).wait()
            pltpu.sync_copy(rows_v, out_hbm.at[pl.ds(off, CHUNK)])
            return 0

        lax.fori_loop(0, n_chunks, body, 0)

    return gather_kernel


def kernel(x, table):
    B, L = x.shape
    N, D = table.shape
    T = B * L
    flat = x.reshape(T)
    out = _make_gather(T, N, D)(table, flat)
    return out.reshape(B, L, D)


# speedup vs baseline: 4.2411x; 1.0724x over previous
import functools

import jax
import jax.numpy as jnp
from jax import lax
from jax.experimental import pallas as pl
from jax.experimental.pallas import tpu as pltpu
from jax.experimental.pallas import tpu_sc as plsc

_NBUF = 2
_CHUNK = 512


def _make_gather(T: int, N: int, D: int):
    info = plsc.get_sparse_core_info()
    NC, NS = info.num_cores, info.num_subcores
    NW = NC * NS
    assert T % NW == 0
    per_w = T // NW
    assert per_w % (_CHUNK * _NBUF) == 0
    n_groups = per_w // (_CHUNK * _NBUF)

    mesh = plsc.VectorSubcoreMesh(core_axis_name="c", subcore_axis_name="s")

    @functools.partial(
        pl.kernel,
        mesh=mesh,
        out_type=jax.ShapeDtypeStruct((T, D), jnp.float32),
        scratch_types=[
            pltpu.VMEM((_NBUF, _CHUNK), jnp.int32),
            pltpu.VMEM((_NBUF, _CHUNK, D), jnp.float32),
            pltpu.SemaphoreType.DMA((_NBUF,)),
            pltpu.SemaphoreType.DMA((_NBUF,)),
        ],
        compiler_params=pltpu.CompilerParams(use_tc_tiling_on_sc=False),
    )
    def gather_kernel(table_hbm, idx_hbm, out_hbm, idx_v, rows_v, sem_g, sem_s):
        wid = lax.axis_index("s") * NC + lax.axis_index("c")
        base = wid * per_w

        def group(g, _):
            for b in range(_NBUF):
                off = base + (g * _NBUF + b) * _CHUNK

                @pl.when(g > 0)
                def _drain():
                    pltpu.make_async_copy(
                        rows_v.at[b], out_hbm.at[pl.ds(off, _CHUNK)], sem_s.at[b]
                    ).wait()

                pltpu.sync_copy(idx_hbm.at[pl.ds(off, _CHUNK)], idx_v.at[b])
                pltpu.async_copy(table_hbm.at[idx_v.at[b]], rows_v.at[b], sem_g.at[b])
            for b in range(_NBUF):
                off = base + (g * _NBUF + b) * _CHUNK
                pltpu.make_async_copy(
                    table_hbm.at[idx_v.at[b]], rows_v.at[b], sem_g.at[b]
                ).wait()
                pltpu.async_copy(rows_v.at[b], out_hbm.at[pl.ds(off, _CHUNK)], sem_s.at[b])
            return 0

        lax.fori_loop(0, n_groups, group, 0)
        for b in range(_NBUF):
            off = base + ((n_groups - 1) * _NBUF + b) * _CHUNK
            pltpu.make_async_copy(
                rows_v.at[b], out_hbm.at[pl.ds(off, _CHUNK)], sem_s.at[b]
            ).wait()

    return gather_kernel


def kernel(x, table):
    B, L = x.shape
    N, D = table.shape
    T = B * L
    flat = x.reshape(T)
    out = _make_gather(T, N, D)(table, flat)
    return out.reshape(B, L, D)


# speedup vs baseline: 4.2705x; 1.0069x over previous
import functools

import jax
import jax.numpy as jnp
from jax import lax
from jax.experimental import pallas as pl
from jax.experimental.pallas import tpu as pltpu
from jax.experimental.pallas import tpu_sc as plsc

_NBUF = 2
_R = 4


def _make_gather(B: int, L: int, N: int, D: int):
    info = plsc.get_sparse_core_info()
    NC, NS = info.num_cores, info.num_subcores
    NW = NC * NS
    assert B % NW == 0
    rows_pw = B // NW
    assert rows_pw % (_R * _NBUF) == 0
    n_groups = rows_pw // (_R * _NBUF)

    mesh = plsc.VectorSubcoreMesh(core_axis_name="c", subcore_axis_name="s")

    @functools.partial(
        pl.kernel,
        mesh=mesh,
        out_type=jax.ShapeDtypeStruct((B, L, D), jnp.float32),
        scratch_types=[
            pltpu.VMEM((_NBUF, _R, L), jnp.int32),
            pltpu.VMEM((_NBUF, _R, L, D), jnp.float32),
            pltpu.SemaphoreType.DMA((_NBUF,)),
            pltpu.SemaphoreType.DMA((_NBUF,)),
        ],
        compiler_params=pltpu.CompilerParams(use_tc_tiling_on_sc=False),
    )
    def gather_kernel(table_hbm, idx_hbm, out_hbm, idx_v, rows_v, sem_g, sem_s):
        wid = lax.axis_index("s") * NC + lax.axis_index("c")
        base = wid * rows_pw

        def group(g, _):
            for b in range(_NBUF):
                r0 = base + (g * _NBUF + b) * _R

                @pl.when(g > 0)
                def _drain():
                    pltpu.make_async_copy(
                        rows_v.at[b], out_hbm.at[pl.ds(r0, _R)], sem_s.at[b]
                    ).wait()

                pltpu.sync_copy(idx_hbm.at[pl.ds(r0, _R)], idx_v.at[b])
                for r in range(_R):
                    pltpu.async_copy(
                        table_hbm.at[idx_v.at[b, r]], rows_v.at[b, r], sem_g.at[b]
                    )
            for b in range(_NBUF):
                r0 = base + (g * _NBUF + b) * _R
                for r in range(_R):
                    pltpu.make_async_copy(
                        table_hbm.at[idx_v.at[b, r]], rows_v.at[b, r], sem_g.at[b]
                    ).wait()
                pltpu.async_copy(rows_v.at[b], out_hbm.at[pl.ds(r0, _R)], sem_s.at[b])
            return 0

        lax.fori_loop(0, n_groups, group, 0)
        for b in range(_NBUF):
            r0 = base + ((n_groups - 1) * _NBUF + b) * _R
            pltpu.make_async_copy(
                rows_v.at[b], out_hbm.at[pl.ds(r0, _R)], sem_s.at[b]
            ).wait()

    return gather_kernel


def kernel(x, table):
    B, L = x.shape
    N, D = table.shape
    return _make_gather(B, L, N, D)(table, x)
